# baseline (speedup 1.0000x reference)
import jax
import jax.numpy as jnp
from jax import lax
from jax.experimental import pallas as pl
from jax.experimental.pallas import tpu as pltpu
from jax.experimental.pallas import tpu_sc as plsc

_NC, _NS, _L = 2, 16, 16
_NW = _NC * _NS
_SLABS = 4 * 4 * 64
_SPW = _SLABS // _NW


def _sc_body(x_hbm, upd_hbm, out_hbm, u_v, bufs, sem_l, sem_s):
    wid = lax.axis_index("s") * _NC + lax.axis_index("c")
    b = wid // 8
    c = (wid % 8) // 2
    d0 = (wid % 2) * _SPW
    _NB, _AH, _NCH = 14, 5, 4 * _SPW

    def _src(k):
        return x_hbm.at[b, c, d0 + k // 4, :, pl.ds((k % 4) * 512, 512)]

    def _dst(k):
        return out_hbm.at[b, c, d0 + k // 4, :, pl.ds((k % 4) * 512, 512)]

    loads, stores = {}, {}
    for k in range(_AH):
        loads[k] = pltpu.async_copy(_src(k), bufs.at[k % _NB], sem_l.at[k % _NB])
    pltpu.sync_copy(upd_hbm, u_v)
    io = lax.iota(jnp.int32, _L)
    lane_m = (io >= 1) & (io < 4)
    base = (b * 6).astype(jnp.int32)
    uvecs = []
    for col in (0, 1):
        fidx = jnp.where(
            io == 1, base + 2 + col,
            jnp.where(io == 2, base + 0 + col,
                      jnp.where(io == 3, base + 4 + col, 0)))
        g = plsc.load_gather(u_v, [fidx.astype(jnp.int32)])
        uvecs.append(jnp.where(lane_m, g, 0.0))
    for k in range(_NCH):
        kb = k % _NB
        if k + _AH < _NCH:
            nb = (k + _AH) % _NB
            if k + _AH >= _NB:
                stores[k + _AH - _NB].wait()
            loads[k + _AH] = pltpu.async_copy(
                _src(k + _AH), bufs.at[nb], sem_l.at[nb])
        loads[k].wait()
        if k % 4 == 0:
            for row, uv in ((2, uvecs[0]), (3, uvecs[1])):
                bufs[kb, row, pl.ds(0, _L)] = bufs[kb, row, pl.ds(0, _L)] + uv
        stores[k] = pltpu.async_copy(bufs.at[kb], _dst(k), sem_s.at[kb])
    for k in range(_NCH - _NB + _AH, _NCH):
        stores[k].wait()


def kernel(x, update):
    xp = jnp.transpose(x, (0, 1, 2, 4, 3))
    upd = update.reshape(-1)
    outp = pl.kernel(
        _sc_body,
        out_type=jax.ShapeDtypeStruct(xp.shape, xp.dtype),
        mesh=plsc.VectorSubcoreMesh(
            core_axis_name="c", subcore_axis_name="s",
            num_cores=_NC, num_subcores=_NS),
        compiler_params=pltpu.CompilerParams(
            needs_layout_passes=False, use_tc_tiling_on_sc=True),
        scratch_types=[
            pltpu.VMEM((24,), jnp.float32),
            pltpu.VMEM((14, 16, 512), jnp.float32),
            pltpu.SemaphoreType.DMA((14,)),
            pltpu.SemaphoreType.DMA((14,)),
        ],
    )(xp, upd)
    return jnp.transpose(outp, (0, 1, 2, 4, 3))


# speedup vs baseline: 1.0297x; 1.0297x over previous
import jax
import jax.numpy as jnp
from jax import lax
from jax.experimental import pallas as pl
from jax.experimental.pallas import tpu as pltpu
from jax.experimental.pallas import tpu_sc as plsc

_NC, _NS, _L = 2, 16, 16
_NW = _NC * _NS
_SLABS = 4 * 4 * 64
_SPW = _SLABS // _NW


def _sc_body(x_hbm, upd_hbm, out_hbm, u_v, bufs, sem_l, sem_s):
    wid = lax.axis_index("s") * _NC + lax.axis_index("c")
    b = wid // 8
    c = (wid % 8) // 2
    d0 = (wid % 2) * _SPW
    _NB, _AH, _NCH = 7, 3, 2 * _SPW

    def _src(k):
        return x_hbm.at[b, c, d0 + k // 2, :, pl.ds((k % 2) * 1024, 1024)]

    def _dst(k):
        return out_hbm.at[b, c, d0 + k // 2, :, pl.ds((k % 2) * 1024, 1024)]

    loads, stores = {}, {}
    for k in range(_AH):
        loads[k] = pltpu.async_copy(_src(k), bufs.at[k % _NB], sem_l.at[k % _NB])
    pltpu.sync_copy(upd_hbm, u_v)
    io = lax.iota(jnp.int32, _L)
    lane_m = (io >= 1) & (io < 4)
    base = (b * 6).astype(jnp.int32)
    uvecs = []
    for col in (0, 1):
        fidx = jnp.where(
            io == 1, base + 2 + col,
            jnp.where(io == 2, base + 0 + col,
                      jnp.where(io == 3, base + 4 + col, 0)))
        g = plsc.load_gather(u_v, [fidx.astype(jnp.int32)])
        uvecs.append(jnp.where(lane_m, g, 0.0))
    for k in range(_NCH):
        kb = k % _NB
        if k + _AH < _NCH:
            nb = (k + _AH) % _NB
            if k + _AH >= _NB:
                stores[k + _AH - _NB].wait()
            loads[k + _AH] = pltpu.async_copy(
                _src(k + _AH), bufs.at[nb], sem_l.at[nb])
        loads[k].wait()
        if k % 2 == 0:
            for row, uv in ((2, uvecs[0]), (3, uvecs[1])):
                bufs[kb, row, pl.ds(0, _L)] = bufs[kb, row, pl.ds(0, _L)] + uv
        stores[k] = pltpu.async_copy(bufs.at[kb], _dst(k), sem_s.at[kb])
    for k in range(_NCH - _NB + _AH, _NCH):
        stores[k].wait()


def kernel(x, update):
    xp = jnp.transpose(x, (0, 1, 2, 4, 3))
    upd = update.reshape(-1)
    outp = pl.kernel(
        _sc_body,
        out_type=jax.ShapeDtypeStruct(xp.shape, xp.dtype),
        mesh=plsc.VectorSubcoreMesh(
            core_axis_name="c", subcore_axis_name="s",
            num_cores=_NC, num_subcores=_NS),
        compiler_params=pltpu.CompilerParams(
            needs_layout_passes=False, use_tc_tiling_on_sc=True),
        scratch_types=[
            pltpu.VMEM((24,), jnp.float32),
            pltpu.VMEM((7, 16, 1024), jnp.float32),
            pltpu.SemaphoreType.DMA((7,)),
            pltpu.SemaphoreType.DMA((7,)),
        ],
    )(xp, upd)
    return jnp.transpose(outp, (0, 1, 2, 4, 3))
